# baseline (speedup 1.0000x reference)
import jax
import jax.numpy as jnp
from jax.experimental import pallas as pl
from jax.experimental.pallas import tpu as pltpu

_N = 10000
_IN_DIM = 128
_HID = 128
_ZDIM = 64
_BM = 400


def _mmT(a, b):
    return jax.lax.dot_general(
        a, b, (((1,), (1,)), ((), ())), preferred_element_type=jnp.float32)


def _vgae_body(x_ref, adj0_ref, adj1_ref, W1_ref, b1_ref, Wm1_ref, bm1_ref,
               Wm2_ref, bm2_ref, Ws1_ref, bs1_ref, Ws2_ref, bs2_ref,
               mean_ref, std_ref, xw_ref):
    i = pl.program_id(0)

    @pl.when(i == 0)
    def _():
        xw_ref[...] = (_mmT(x_ref[...], W1_ref[...])
                       + b1_ref[...]).astype(jnp.bfloat16)

    h0 = jnp.dot(adj0_ref[...].astype(jnp.bfloat16), xw_ref[...],
                 preferred_element_type=jnp.float32)
    h1 = jnp.dot(adj1_ref[...].astype(jnp.bfloat16), xw_ref[...],
                 preferred_element_type=jnp.float32)
    h = jnp.maximum(jnp.concatenate([h0, h1], axis=0), 0.0)
    hm = jnp.maximum(_mmT(h, Wm1_ref[...]) + bm1_ref[...], 0.0)
    mean_ref[...] = _mmT(hm, Wm2_ref[...]) + bm2_ref[...]
    hs = jnp.maximum(_mmT(h, Ws1_ref[...]) + bs1_ref[...], 0.0)
    std_ref[...] = jax.nn.softplus(_mmT(hs, Ws2_ref[...]) + bs2_ref[...])


def kernel(x, adj, W1, b1, Wm1, bm1, Wm2, bm2, Ws1, bs1, Ws2, bs2):
    full = lambda shape: pl.BlockSpec(shape, lambda i: (0, 0))
    grid = (_N // _BM,)
    mean, std = pl.pallas_call(
        _vgae_body,
        grid=grid,
        in_specs=[
            full((_N, _IN_DIM)),
            pl.BlockSpec((_BM // 2, _N), lambda i: (2 * i, 0)),
            pl.BlockSpec((_BM // 2, _N), lambda i: (2 * i + 1, 0)),
            full((_HID, _IN_DIM)),
            full((1, _HID)),
            full((_ZDIM, _HID)),
            full((1, _ZDIM)),
            full((_ZDIM, _ZDIM)),
            full((1, _ZDIM)),
            full((_ZDIM, _HID)),
            full((1, _ZDIM)),
            full((_ZDIM, _ZDIM)),
            full((1, _ZDIM)),
        ],
        out_specs=[
            pl.BlockSpec((_BM, _ZDIM), lambda i: (i, 0)),
            pl.BlockSpec((_BM, _ZDIM), lambda i: (i, 0)),
        ],
        out_shape=[
            jax.ShapeDtypeStruct((_N, _ZDIM), jnp.float32),
            jax.ShapeDtypeStruct((_N, _ZDIM), jnp.float32),
        ],
        scratch_shapes=[pltpu.VMEM((_N, _HID), jnp.bfloat16)],
    )(x, adj, adj, W1, b1.reshape(1, _HID), Wm1, bm1.reshape(1, _ZDIM),
      Wm2, bm2.reshape(1, _ZDIM), Ws1, bs1.reshape(1, _ZDIM),
      Ws2, bs2.reshape(1, _ZDIM))
    return (mean, mean, std)


# speedup vs baseline: 1.0173x; 1.0173x over previous
import jax
import jax.numpy as jnp
from jax.experimental import pallas as pl
from jax.experimental.pallas import tpu as pltpu

_N = 10000
_IN_DIM = 128
_HID = 128
_ZDIM = 64
_BM = 400


def _mmT(a, b):
    return jax.lax.dot_general(
        a, b, (((1,), (1,)), ((), ())), preferred_element_type=jnp.float32)


def _vgae_body(x_ref, adj_ref, W1_ref, b1_ref, Wm1_ref, bm1_ref, Wm2_ref,
               bm2_ref, Ws1_ref, bs1_ref, Ws2_ref, bs2_ref,
               mean_ref, std_ref, xw_ref):
    i = pl.program_id(0)

    @pl.when(i == 0)
    def _():
        xw_ref[...] = (_mmT(x_ref[...], W1_ref[...])
                       + b1_ref[...]).astype(jnp.bfloat16)

    h = jnp.maximum(
        jnp.dot(adj_ref[...].astype(jnp.bfloat16), xw_ref[...],
                preferred_element_type=jnp.float32),
        0.0)
    hm = jnp.maximum(_mmT(h, Wm1_ref[...]) + bm1_ref[...], 0.0)
    mean_ref[...] = _mmT(hm, Wm2_ref[...]) + bm2_ref[...]
    hs = jnp.maximum(_mmT(h, Ws1_ref[...]) + bs1_ref[...], 0.0)
    std_ref[...] = jax.nn.softplus(_mmT(hs, Ws2_ref[...]) + bs2_ref[...])


def kernel(x, adj, W1, b1, Wm1, bm1, Wm2, bm2, Ws1, bs1, Ws2, bs2):
    full = lambda shape: pl.BlockSpec(shape, lambda i: (0, 0))
    grid = (_N // _BM,)
    mean, std = pl.pallas_call(
        _vgae_body,
        grid=grid,
        in_specs=[
            full((_N, _IN_DIM)),
            pl.BlockSpec((_BM, _N), lambda i: (i, 0)),
            full((_HID, _IN_DIM)),
            full((1, _HID)),
            full((_ZDIM, _HID)),
            full((1, _ZDIM)),
            full((_ZDIM, _ZDIM)),
            full((1, _ZDIM)),
            full((_ZDIM, _HID)),
            full((1, _ZDIM)),
            full((_ZDIM, _ZDIM)),
            full((1, _ZDIM)),
        ],
        out_specs=[
            pl.BlockSpec((_BM, _ZDIM), lambda i: (i, 0)),
            pl.BlockSpec((_BM, _ZDIM), lambda i: (i, 0)),
        ],
        out_shape=[
            jax.ShapeDtypeStruct((_N, _ZDIM), jnp.float32),
            jax.ShapeDtypeStruct((_N, _ZDIM), jnp.float32),
        ],
        scratch_shapes=[pltpu.VMEM((_N, _HID), jnp.bfloat16)],
    )(x, adj, W1, b1.reshape(1, _HID), Wm1, bm1.reshape(1, _ZDIM),
      Wm2, bm2.reshape(1, _ZDIM), Ws1, bs1.reshape(1, _ZDIM),
      Ws2, bs2.reshape(1, _ZDIM))
    return (mean, mean, std)
